# baseline (speedup 1.0000x reference)
import dataclasses
import functools

import jax
import jax.numpy as jnp
from jax import lax
from jax.experimental import pallas as pl
from jax.experimental.pallas import tpu as pltpu
from jax.experimental.pallas import tpu_sc as plsc

_N = 10000
_E = 320000
_D = 128
_NC = 2
_NS = 16
_NW = _NC * _NS
_CHUNK = 128
_CH = 80
_CHS = 16
_STAGES = _CH // _CHS
_EPW = _CH * _CHUNK
_EPAD = _NW * _EPW
_NPAD = 10240
_RPT = _NPAD // _NS
_BLK = 1024
_NBLK = _NPAD // _BLK

_mesh = plsc.VectorSubcoreMesh(core_axis_name="c", subcore_axis_name="s")

_sc_params = pltpu.CompilerParams()
if "needs_layout_passes" in pltpu.CompilerParams.__dataclass_fields__:
    _sc_params = dataclasses.replace(_sc_params, needs_layout_passes=False)


@functools.partial(
    pl.kernel,
    out_type=jax.ShapeDtypeStruct((2, _NW, _NPAD), jnp.float32),
    mesh=_mesh,
    scratch_types=[
        pltpu.VMEM((_CH, _CHUNK), jnp.int32),
        pltpu.VMEM((_CH, _CHUNK), jnp.int32),
        pltpu.VMEM((_NPAD,), jnp.float32),
        pltpu.VMEM((_NPAD,), jnp.float32),
    ],
    compiler_params=_sc_params,
)
def _deg_kernel(row_hbm, col_hbm, out_hbm, rowv, colv, histo, histi):
    cid = lax.axis_index("c")
    sid = lax.axis_index("s")
    wid = cid * _NS + sid

    pltpu.sync_copy(row_hbm.at[wid], rowv)
    pltpu.sync_copy(col_hbm.at[wid], colv)

    zeros16 = jnp.zeros((16,), jnp.float32)

    @pl.loop(0, _NPAD, step=16)
    def _zero(i):
        histo[pl.ds(i, 16)] = zeros16
        histi[pl.ds(i, 16)] = zeros16

    ones16 = jnp.ones((16,), jnp.float32)

    @pl.loop(0, _CH)
    def _chunk(j):
        @pl.loop(0, _CHUNK, step=16)
        def _vec(k):
            plsc.addupdate_scatter(histo, [rowv[j, pl.ds(k, 16)]], ones16)
            plsc.addupdate_scatter(histi, [colv[j, pl.ds(k, 16)]], ones16)

    pltpu.sync_copy(histo, out_hbm.at[0, wid])
    pltpu.sync_copy(histi, out_hbm.at[1, wid])


@functools.partial(
    pl.kernel,
    out_type=jax.ShapeDtypeStruct((_NC, _NPAD, _D), jnp.float32),
    mesh=_mesh,
    scratch_types=[
        pltpu.VMEM((_CHS, _CHUNK), jnp.int32),
        pltpu.VMEM((_CHS, _CHUNK), jnp.int32),
        pltpu.VMEM((_CHUNK, _D), jnp.float32),
        pltpu.VMEM((_CHUNK, _D), jnp.float32),
        pltpu.VMEM_SHARED((_NPAD, _D), jnp.float32),
        pltpu.SemaphoreType.DMA,
        pltpu.SemaphoreType.DMA,
        pltpu.SemaphoreType.DMA,
        pltpu.SemaphoreType.DMA,
    ],
    compiler_params=_sc_params,
)
def _agg_kernel(t_hbm, row_hbm, col_hbm, out_hbm, rowv, colv, bufa, bufb,
                acc, sema, semb, semsa, semsb):
    cid = lax.axis_index("c")
    sid = lax.axis_index("s")
    wid = cid * _NS + sid

    zeros16 = jnp.zeros((16,), jnp.float32)

    @pl.loop(0, _CHUNK)
    def _zrow(r):
        @pl.loop(0, _D, step=16)
        def _zcol(k):
            bufa[r, pl.ds(k, 16)] = zeros16

    @pl.loop(0, _RPT, step=_CHUNK)
    def _zacc(r0):
        pltpu.sync_copy(bufa, acc.at[pl.ds(sid * _RPT + r0, _CHUNK)])

    plsc.subcore_barrier()

    @pl.loop(0, _STAGES)
    def _stage(s):
        pltpu.sync_copy(row_hbm.at[wid, pl.ds(s * _CHS, _CHS)], rowv)
        pltpu.sync_copy(col_hbm.at[wid, pl.ds(s * _CHS, _CHS)], colv)

        pltpu.async_copy(t_hbm.at[colv.at[0]], bufa, sema)
        pltpu.async_copy(t_hbm.at[colv.at[1]], bufb, semb)

        @pl.loop(0, _CHS - 2, step=2)
        def _body(j):
            pltpu.make_async_copy(t_hbm.at[colv.at[j]], bufa, sema).wait()
            pltpu.async_copy(bufa, acc.at[rowv.at[j]], semsa, add=True)
            pltpu.make_async_copy(t_hbm.at[colv.at[j + 1]], bufb, semb).wait()
            pltpu.async_copy(bufb, acc.at[rowv.at[j + 1]], semsb, add=True)
            pltpu.make_async_copy(bufa, acc.at[rowv.at[j]], semsa).wait()
            pltpu.async_copy(t_hbm.at[colv.at[j + 2]], bufa, sema)
            pltpu.make_async_copy(bufb, acc.at[rowv.at[j + 1]], semsb).wait()
            pltpu.async_copy(t_hbm.at[colv.at[j + 3]], bufb, semb)

        pltpu.make_async_copy(t_hbm.at[colv.at[_CHS - 2]], bufa, sema).wait()
        pltpu.async_copy(bufa, acc.at[rowv.at[_CHS - 2]], semsa, add=True)
        pltpu.make_async_copy(t_hbm.at[colv.at[_CHS - 1]], bufb, semb).wait()
        pltpu.async_copy(bufb, acc.at[rowv.at[_CHS - 1]], semsb, add=True)
        pltpu.make_async_copy(bufa, acc.at[rowv.at[_CHS - 2]], semsa).wait()
        pltpu.make_async_copy(bufb, acc.at[rowv.at[_CHS - 1]], semsb).wait()

    plsc.subcore_barrier()

    pltpu.sync_copy(acc.at[pl.ds(sid * _RPT, _RPT)],
                    out_hbm.at[cid, pl.ds(sid * _RPT, _RPT)])


def _mm2_body(x_ref, w_ref, c_ref, t_ref, h_ref):
    xb = x_ref[...]
    dn = (((1,), (1,)), ((), ()))
    t_ref[...] = lax.dot_general(xb, w_ref[...], dn,
                                 preferred_element_type=jnp.float32)
    h_ref[...] = lax.dot_general(xb, c_ref[...], dn,
                                 preferred_element_type=jnp.float32)


def _mm2(x, w, c):
    return pl.pallas_call(
        _mm2_body,
        grid=(_NBLK,),
        in_specs=[
            pl.BlockSpec((_BLK, _D), lambda i: (i, 0)),
            pl.BlockSpec((_D, _D), lambda i: (0, 0)),
            pl.BlockSpec((_D, _D), lambda i: (0, 0)),
        ],
        out_specs=[
            pl.BlockSpec((_BLK, _D), lambda i: (i, 0)),
            pl.BlockSpec((_BLK, _D), lambda i: (i, 0)),
        ],
        out_shape=[
            jax.ShapeDtypeStruct((_NPAD, _D), jnp.float32),
            jax.ShapeDtypeStruct((_NPAD, _D), jnp.float32),
        ],
    )(x, w, c)


def _degnorm_body(dp_ref, d_ref):
    for h in range(2):
        deg = dp_ref[h, 0]
        for w in range(1, _NW):
            deg = deg + dp_ref[h, w]
        d_ref[h] = jnp.where(deg > 0.0, lax.rsqrt(deg), 0.0)


def _degnorm(dp):
    return pl.pallas_call(
        _degnorm_body,
        out_shape=jax.ShapeDtypeStruct((2, _NPAD // _D, _D), jnp.float32),
    )(dp)


def _scale_body(t_ref, d_ref, o_ref):
    o_ref[...] = d_ref[...] * t_ref[...]


def _scale(t, d_col):
    return pl.pallas_call(
        _scale_body,
        grid=(_NBLK,),
        in_specs=[
            pl.BlockSpec((_BLK, _D), lambda i: (i, 0)),
            pl.BlockSpec((_BLK, 1), lambda i: (i, 0)),
        ],
        out_specs=pl.BlockSpec((_BLK, _D), lambda i: (i, 0)),
        out_shape=jax.ShapeDtypeStruct((_NPAD, _D), jnp.float32),
    )(t, d_col)


def _layer2_body(acc_ref, do_ref, di_ref, hd_ref, w_ref, c_ref,
                 t2s_ref, hd2_ref):
    h1 = jnp.maximum(do_ref[...] * (acc_ref[0] + acc_ref[1]) + hd_ref[...],
                     0.0)
    dn = (((1,), (1,)), ((), ()))
    t2s_ref[...] = di_ref[...] * lax.dot_general(
        h1, w_ref[...], dn, preferred_element_type=jnp.float32)
    hd2_ref[...] = lax.dot_general(h1, c_ref[...], dn,
                                   preferred_element_type=jnp.float32)


def _layer2(acc, d_out_col, d_in_col, hdir1, w2, c2):
    return pl.pallas_call(
        _layer2_body,
        grid=(_NBLK,),
        in_specs=[
            pl.BlockSpec((_NC, _BLK, _D), lambda i: (0, i, 0)),
            pl.BlockSpec((_BLK, 1), lambda i: (i, 0)),
            pl.BlockSpec((_BLK, 1), lambda i: (i, 0)),
            pl.BlockSpec((_BLK, _D), lambda i: (i, 0)),
            pl.BlockSpec((_D, _D), lambda i: (0, 0)),
            pl.BlockSpec((_D, _D), lambda i: (0, 0)),
        ],
        out_specs=[
            pl.BlockSpec((_BLK, _D), lambda i: (i, 0)),
            pl.BlockSpec((_BLK, _D), lambda i: (i, 0)),
        ],
        out_shape=[
            jax.ShapeDtypeStruct((_NPAD, _D), jnp.float32),
            jax.ShapeDtypeStruct((_NPAD, _D), jnp.float32),
        ],
    )(acc, d_out_col, d_in_col, hdir1, w2, c2)


def _final_body(acc_ref, do_ref, hd_ref, o_ref):
    o_ref[...] = do_ref[...] * (acc_ref[0] + acc_ref[1]) + hd_ref[...]


def _final(acc, d_out_col, hdir2):
    return pl.pallas_call(
        _final_body,
        grid=(_NBLK,),
        in_specs=[
            pl.BlockSpec((_NC, _BLK, _D), lambda i: (0, i, 0)),
            pl.BlockSpec((_BLK, 1), lambda i: (i, 0)),
            pl.BlockSpec((_BLK, _D), lambda i: (i, 0)),
        ],
        out_specs=pl.BlockSpec((_BLK, _D), lambda i: (i, 0)),
        out_shape=jax.ShapeDtypeStruct((_NPAD, _D), jnp.float32),
    )(acc, d_out_col, hdir2)


def kernel(x, edge_index, W1, Wdir1, alpha1, beta1, W2, Wdir2, alpha2, beta2):
    row = edge_index[0]
    col = edge_index[1]
    pad = _N + (jnp.arange(_EPAD - _E, dtype=jnp.int32) % (_NPAD - _N))
    row_p = jnp.concatenate([row, pad]).reshape(_NW, _CH, _CHUNK)
    col_p = jnp.concatenate([col, pad]).reshape(_NW, _CH, _CHUNK)

    x_pad = jnp.pad(x, ((0, _NPAD - _N), (0, 0)))

    c1 = alpha1 * W1 + beta1 * Wdir1
    c2 = alpha2 * W2 + beta2 * Wdir2

    dp = _deg_kernel(row_p, col_p)
    t1, hdir1 = _mm2(x_pad, W1, c1)

    d = _degnorm(dp.reshape(2, _NW, _NPAD // _D, _D))
    d_out_col = d[0].reshape(_NPAD, 1)
    d_in_col = d[1].reshape(_NPAD, 1)

    t1s = _scale(t1, d_in_col)
    acc1 = _agg_kernel(t1s, row_p, col_p)

    t2s, hdir2 = _layer2(acc1, d_out_col, d_in_col, hdir1, W2, c2)
    acc2 = _agg_kernel(t2s, row_p, col_p)

    out = _final(acc2, d_out_col, hdir2)
    return out[:_N]


# speedup vs baseline: 1.1929x; 1.1929x over previous
import dataclasses
import functools

import jax
import jax.numpy as jnp
from jax import lax
from jax.experimental import pallas as pl
from jax.experimental.pallas import tpu as pltpu
from jax.experimental.pallas import tpu_sc as plsc

_N = 10000
_E = 320000
_D = 128
_NC = 2
_NS = 16
_NW = _NC * _NS
_CHUNK = 128
_CH = 80
_CHS = 16
_STAGES = _CH // _CHS
_EPW = _CH * _CHUNK
_EPAD = _NW * _EPW
_NPAD = 10240
_RPT = _NPAD // _NS
_BLK = 1024
_NBLK = _NPAD // _BLK

_mesh = plsc.VectorSubcoreMesh(core_axis_name="c", subcore_axis_name="s")

_sc_params = pltpu.CompilerParams()
if "needs_layout_passes" in pltpu.CompilerParams.__dataclass_fields__:
    _sc_params = dataclasses.replace(_sc_params, needs_layout_passes=False)


@functools.partial(
    pl.kernel,
    out_type=jax.ShapeDtypeStruct((2, _NW, _NPAD), jnp.float32),
    mesh=_mesh,
    scratch_types=[
        pltpu.VMEM((_CH, _CHUNK), jnp.int32),
        pltpu.VMEM((_CH, _CHUNK), jnp.int32),
        pltpu.VMEM((_NPAD,), jnp.float32),
        pltpu.VMEM((_NPAD,), jnp.float32),
    ],
    compiler_params=_sc_params,
)
def _deg_kernel(row_hbm, col_hbm, out_hbm, rowv, colv, histo, histi):
    cid = lax.axis_index("c")
    sid = lax.axis_index("s")
    wid = cid * _NS + sid

    pltpu.sync_copy(row_hbm.at[wid], rowv)
    pltpu.sync_copy(col_hbm.at[wid], colv)

    zeros16 = jnp.zeros((16,), jnp.float32)

    @pl.loop(0, _NPAD, step=16)
    def _zero(i):
        histo[pl.ds(i, 16)] = zeros16
        histi[pl.ds(i, 16)] = zeros16

    ones16 = jnp.ones((16,), jnp.float32)

    @pl.loop(0, _CH)
    def _chunk(j):
        @pl.loop(0, _CHUNK, step=16)
        def _vec(k):
            plsc.addupdate_scatter(histo, [rowv[j, pl.ds(k, 16)]], ones16)
            plsc.addupdate_scatter(histi, [colv[j, pl.ds(k, 16)]], ones16)

    pltpu.sync_copy(histo, out_hbm.at[0, wid])
    pltpu.sync_copy(histi, out_hbm.at[1, wid])


@functools.partial(
    pl.kernel,
    out_type=jax.ShapeDtypeStruct((_NC, _NPAD, _D), jnp.float32),
    mesh=_mesh,
    scratch_types=[
        pltpu.VMEM((_CHS, _CHUNK), jnp.int32),
        pltpu.VMEM((_CHS, _CHUNK), jnp.int32),
        pltpu.VMEM((_CHUNK, _D), jnp.float32),
        pltpu.VMEM((_CHUNK, _D), jnp.float32),
        pltpu.VMEM_SHARED((_NPAD, _D), jnp.float32),
        pltpu.SemaphoreType.DMA,
        pltpu.SemaphoreType.DMA,
    ],
    compiler_params=_sc_params,
)
def _agg_kernel(t_hbm, row_hbm, col_hbm, out_hbm, rowv, colv, bufa, bufb,
                acc, sema, semb):
    cid = lax.axis_index("c")
    sid = lax.axis_index("s")
    wid = cid * _NS + sid

    zeros16 = jnp.zeros((16,), jnp.float32)

    @pl.loop(0, _CHUNK)
    def _zrow(r):
        @pl.loop(0, _D, step=16)
        def _zcol(k):
            bufa[r, pl.ds(k, 16)] = zeros16

    @pl.loop(0, _RPT, step=_CHUNK)
    def _zacc(r0):
        pltpu.sync_copy(bufa, acc.at[pl.ds(sid * _RPT + r0, _CHUNK)])

    plsc.subcore_barrier()

    @pl.loop(0, _STAGES)
    def _stage(s):
        pltpu.sync_copy(row_hbm.at[wid, pl.ds(s * _CHS, _CHS)], rowv)
        pltpu.sync_copy(col_hbm.at[wid, pl.ds(s * _CHS, _CHS)], colv)

        pltpu.async_copy(t_hbm.at[colv.at[0]], bufa, sema)

        @pl.loop(0, _CHS - 2, step=2)
        def _body(j):
            pltpu.async_copy(t_hbm.at[colv.at[j + 1]], bufb, semb)
            pltpu.make_async_copy(t_hbm.at[colv.at[j]], bufa, sema).wait()
            pltpu.sync_copy(bufa, acc.at[rowv.at[j]], add=True)
            pltpu.async_copy(t_hbm.at[colv.at[j + 2]], bufa, sema)
            pltpu.make_async_copy(t_hbm.at[colv.at[j + 1]], bufb, semb).wait()
            pltpu.sync_copy(bufb, acc.at[rowv.at[j + 1]], add=True)

        pltpu.async_copy(t_hbm.at[colv.at[_CHS - 1]], bufb, semb)
        pltpu.make_async_copy(t_hbm.at[colv.at[_CHS - 2]], bufa, sema).wait()
        pltpu.sync_copy(bufa, acc.at[rowv.at[_CHS - 2]], add=True)
        pltpu.make_async_copy(t_hbm.at[colv.at[_CHS - 1]], bufb, semb).wait()
        pltpu.sync_copy(bufb, acc.at[rowv.at[_CHS - 1]], add=True)

    plsc.subcore_barrier()

    pltpu.sync_copy(acc.at[pl.ds(sid * _RPT, _RPT)],
                    out_hbm.at[cid, pl.ds(sid * _RPT, _RPT)])


def _mm2_body(x_ref, w_ref, c_ref, di_ref, t_ref, h_ref):
    xb = x_ref[...]
    dn = (((1,), (1,)), ((), ()))
    t_ref[...] = di_ref[...] * lax.dot_general(
        xb, w_ref[...], dn, preferred_element_type=jnp.float32)
    h_ref[...] = lax.dot_general(xb, c_ref[...], dn,
                                 preferred_element_type=jnp.float32)


def _mm2(x, w, c, d_in_col):
    return pl.pallas_call(
        _mm2_body,
        grid=(_NBLK,),
        in_specs=[
            pl.BlockSpec((_BLK, _D), lambda i: (i, 0)),
            pl.BlockSpec((_D, _D), lambda i: (0, 0)),
            pl.BlockSpec((_D, _D), lambda i: (0, 0)),
            pl.BlockSpec((_BLK, 1), lambda i: (i, 0)),
        ],
        out_specs=[
            pl.BlockSpec((_BLK, _D), lambda i: (i, 0)),
            pl.BlockSpec((_BLK, _D), lambda i: (i, 0)),
        ],
        out_shape=[
            jax.ShapeDtypeStruct((_NPAD, _D), jnp.float32),
            jax.ShapeDtypeStruct((_NPAD, _D), jnp.float32),
        ],
    )(x, w, c, d_in_col)


def _degnorm_body(dp_ref, d_ref):
    for h in range(2):
        deg = dp_ref[h, 0]
        for w in range(1, _NW):
            deg = deg + dp_ref[h, w]
        d_ref[h] = jnp.where(deg > 0.0, lax.rsqrt(deg), 0.0)


def _degnorm(dp):
    return pl.pallas_call(
        _degnorm_body,
        out_shape=jax.ShapeDtypeStruct((2, _NPAD // _D, _D), jnp.float32),
    )(dp)


def _scale_body(t_ref, d_ref, o_ref):
    o_ref[...] = d_ref[...] * t_ref[...]


def _scale(t, d_col):
    return pl.pallas_call(
        _scale_body,
        grid=(_NBLK,),
        in_specs=[
            pl.BlockSpec((_BLK, _D), lambda i: (i, 0)),
            pl.BlockSpec((_BLK, 1), lambda i: (i, 0)),
        ],
        out_specs=pl.BlockSpec((_BLK, _D), lambda i: (i, 0)),
        out_shape=jax.ShapeDtypeStruct((_NPAD, _D), jnp.float32),
    )(t, d_col)


def _layer2_body(acc_ref, do_ref, di_ref, hd_ref, w_ref, c_ref,
                 t2s_ref, hd2_ref):
    h1 = jnp.maximum(do_ref[...] * (acc_ref[0] + acc_ref[1]) + hd_ref[...],
                     0.0)
    dn = (((1,), (1,)), ((), ()))
    t2s_ref[...] = di_ref[...] * lax.dot_general(
        h1, w_ref[...], dn, preferred_element_type=jnp.float32)
    hd2_ref[...] = lax.dot_general(h1, c_ref[...], dn,
                                   preferred_element_type=jnp.float32)


def _layer2(acc, d_out_col, d_in_col, hdir1, w2, c2):
    return pl.pallas_call(
        _layer2_body,
        grid=(_NBLK,),
        in_specs=[
            pl.BlockSpec((_NC, _BLK, _D), lambda i: (0, i, 0)),
            pl.BlockSpec((_BLK, 1), lambda i: (i, 0)),
            pl.BlockSpec((_BLK, 1), lambda i: (i, 0)),
            pl.BlockSpec((_BLK, _D), lambda i: (i, 0)),
            pl.BlockSpec((_D, _D), lambda i: (0, 0)),
            pl.BlockSpec((_D, _D), lambda i: (0, 0)),
        ],
        out_specs=[
            pl.BlockSpec((_BLK, _D), lambda i: (i, 0)),
            pl.BlockSpec((_BLK, _D), lambda i: (i, 0)),
        ],
        out_shape=[
            jax.ShapeDtypeStruct((_NPAD, _D), jnp.float32),
            jax.ShapeDtypeStruct((_NPAD, _D), jnp.float32),
        ],
    )(acc, d_out_col, d_in_col, hdir1, w2, c2)


def _final_body(acc_ref, do_ref, hd_ref, o_ref):
    o_ref[...] = do_ref[...] * (acc_ref[0] + acc_ref[1]) + hd_ref[...]


def _final(acc, d_out_col, hdir2):
    return pl.pallas_call(
        _final_body,
        grid=(_NBLK,),
        in_specs=[
            pl.BlockSpec((_NC, _BLK, _D), lambda i: (0, i, 0)),
            pl.BlockSpec((_BLK, 1), lambda i: (i, 0)),
            pl.BlockSpec((_BLK, _D), lambda i: (i, 0)),
        ],
        out_specs=pl.BlockSpec((_BLK, _D), lambda i: (i, 0)),
        out_shape=jax.ShapeDtypeStruct((_NPAD, _D), jnp.float32),
    )(acc, d_out_col, hdir2)


def kernel(x, edge_index, W1, Wdir1, alpha1, beta1, W2, Wdir2, alpha2, beta2):
    row = edge_index[0]
    col = edge_index[1]
    pad = _N + (jnp.arange(_EPAD - _E, dtype=jnp.int32) % (_NPAD - _N))
    row_p = jnp.concatenate([row, pad]).reshape(_NW, _CH, _CHUNK)
    col_p = jnp.concatenate([col, pad]).reshape(_NW, _CH, _CHUNK)

    x_pad = jnp.pad(x, ((0, _NPAD - _N), (0, 0)))

    c1 = alpha1 * W1 + beta1 * Wdir1
    c2 = alpha2 * W2 + beta2 * Wdir2

    dp = _deg_kernel(row_p, col_p)
    d = _degnorm(dp.reshape(2, _NW, _NPAD // _D, _D))
    d_out_col = d[0].reshape(_NPAD, 1)
    d_in_col = d[1].reshape(_NPAD, 1)

    t1s, hdir1 = _mm2(x_pad, W1, c1, d_in_col)
    acc1 = _agg_kernel(t1s, row_p, col_p)

    t2s, hdir2 = _layer2(acc1, d_out_col, d_in_col, hdir1, W2, c2)
    acc2 = _agg_kernel(t2s, row_p, col_p)

    out = _final(acc2, d_out_col, hdir2)
    return out[:_N]


# speedup vs baseline: 1.2170x; 1.0202x over previous
import dataclasses
import functools

import jax
import jax.numpy as jnp
from jax import lax
from jax.experimental import pallas as pl
from jax.experimental.pallas import tpu as pltpu
from jax.experimental.pallas import tpu_sc as plsc

_N = 10000
_E = 320000
_D = 128
_NC = 2
_NS = 16
_NW = _NC * _NS
_CHUNK = 125
_CH = 80
_EPW = _CH * _CHUNK
_CHS = 16
_STAGES = _CH // _CHS
_SLAB_STRIDE = 624
_SLAB = 640
_BLK = 1000
_NBLK = _N // _BLK

_mesh = plsc.VectorSubcoreMesh(core_axis_name="c", subcore_axis_name="s")

_sc_params = pltpu.CompilerParams()
if "needs_layout_passes" in pltpu.CompilerParams.__dataclass_fields__:
    _sc_params = dataclasses.replace(_sc_params, needs_layout_passes=False)


@functools.partial(
    pl.kernel,
    out_type=jax.ShapeDtypeStruct((2, _NW, _N), jnp.float32),
    mesh=_mesh,
    scratch_types=[
        pltpu.VMEM((_EPW,), jnp.int32),
        pltpu.VMEM((_EPW,), jnp.int32),
        pltpu.VMEM((_N,), jnp.float32),
        pltpu.VMEM((_N,), jnp.float32),
    ],
    compiler_params=_sc_params,
)
def _deg_kernel(row_hbm, col_hbm, out_hbm, rowv, colv, histo, histi):
    cid = lax.axis_index("c")
    sid = lax.axis_index("s")
    wid = cid * _NS + sid

    pltpu.sync_copy(row_hbm.at[pl.ds(wid * _EPW, _EPW)], rowv)
    pltpu.sync_copy(col_hbm.at[pl.ds(wid * _EPW, _EPW)], colv)

    zeros16 = jnp.zeros((16,), jnp.float32)

    @pl.loop(0, _N, step=16)
    def _zero(i):
        histo[pl.ds(i, 16)] = zeros16
        histi[pl.ds(i, 16)] = zeros16

    ones16 = jnp.ones((16,), jnp.float32)

    @pl.loop(0, _EPW, step=16)
    def _vec(k):
        plsc.addupdate_scatter(histo, [rowv[pl.ds(k, 16)]], ones16)
        plsc.addupdate_scatter(histi, [colv[pl.ds(k, 16)]], ones16)

    pltpu.sync_copy(histo, out_hbm.at[0, wid])
    pltpu.sync_copy(histi, out_hbm.at[1, wid])


@functools.partial(
    pl.kernel,
    out_type=jax.ShapeDtypeStruct((_NC, _N, _D), jnp.float32),
    mesh=_mesh,
    scratch_types=[
        pltpu.VMEM((_CHS, _CHUNK), jnp.int32),
        pltpu.VMEM((_CHS, _CHUNK), jnp.int32),
        pltpu.VMEM((_CHUNK, _D), jnp.float32),
        pltpu.VMEM((_CHUNK, _D), jnp.float32),
        pltpu.VMEM_SHARED((_N, _D), jnp.float32),
        pltpu.SemaphoreType.DMA,
        pltpu.SemaphoreType.DMA,
    ],
    compiler_params=_sc_params,
)
def _agg_kernel(t_hbm, row_hbm, col_hbm, out_hbm, rowv, colv, bufa, bufb,
                acc, sema, semb):
    cid = lax.axis_index("c")
    sid = lax.axis_index("s")
    wid = cid * _NS + sid

    zeros16 = jnp.zeros((16,), jnp.float32)

    @pl.loop(0, _CHUNK)
    def _zrow(r):
        @pl.loop(0, _D, step=16)
        def _zcol(k):
            bufa[r, pl.ds(k, 16)] = zeros16

    @pl.loop(0, _SLAB, step=64)
    def _zacc(r0):
        pltpu.sync_copy(bufa.at[pl.ds(0, 64)],
                        acc.at[pl.ds(sid * _SLAB_STRIDE + r0, 64)])

    plsc.subcore_barrier()

    @pl.loop(0, _STAGES)
    def _stage(s):
        pltpu.sync_copy(row_hbm.at[wid, pl.ds(s * _CHS, _CHS)], rowv)
        pltpu.sync_copy(col_hbm.at[wid, pl.ds(s * _CHS, _CHS)], colv)

        pltpu.async_copy(t_hbm.at[colv.at[0]], bufa, sema)

        @pl.loop(0, _CHS - 2, step=2)
        def _body(j):
            pltpu.async_copy(t_hbm.at[colv.at[j + 1]], bufb, semb)
            pltpu.make_async_copy(t_hbm.at[colv.at[j]], bufa, sema).wait()
            pltpu.sync_copy(bufa, acc.at[rowv.at[j]], add=True)
            pltpu.async_copy(t_hbm.at[colv.at[j + 2]], bufa, sema)
            pltpu.make_async_copy(t_hbm.at[colv.at[j + 1]], bufb, semb).wait()
            pltpu.sync_copy(bufb, acc.at[rowv.at[j + 1]], add=True)

        pltpu.async_copy(t_hbm.at[colv.at[_CHS - 1]], bufb, semb)
        pltpu.make_async_copy(t_hbm.at[colv.at[_CHS - 2]], bufa, sema).wait()
        pltpu.sync_copy(bufa, acc.at[rowv.at[_CHS - 2]], add=True)
        pltpu.make_async_copy(t_hbm.at[colv.at[_CHS - 1]], bufb, semb).wait()
        pltpu.sync_copy(bufb, acc.at[rowv.at[_CHS - 1]], add=True)

    plsc.subcore_barrier()

    pltpu.sync_copy(acc.at[pl.ds(sid * _SLAB_STRIDE, _SLAB)],
                    out_hbm.at[cid, pl.ds(sid * _SLAB_STRIDE, _SLAB)])


def _mm2_body(x_ref, w_ref, c_ref, di_ref, t_ref, h_ref):
    xb = x_ref[...]
    dn = (((1,), (1,)), ((), ()))
    t_ref[...] = di_ref[...] * lax.dot_general(
        xb, w_ref[...], dn, preferred_element_type=jnp.float32)
    h_ref[...] = lax.dot_general(xb, c_ref[...], dn,
                                 preferred_element_type=jnp.float32)


def _mm2(x, w, c, d_in_col):
    return pl.pallas_call(
        _mm2_body,
        grid=(_NBLK,),
        in_specs=[
            pl.BlockSpec((_BLK, _D), lambda i: (i, 0)),
            pl.BlockSpec((_D, _D), lambda i: (0, 0)),
            pl.BlockSpec((_D, _D), lambda i: (0, 0)),
            pl.BlockSpec((_BLK, 1), lambda i: (i, 0)),
        ],
        out_specs=[
            pl.BlockSpec((_BLK, _D), lambda i: (i, 0)),
            pl.BlockSpec((_BLK, _D), lambda i: (i, 0)),
        ],
        out_shape=[
            jax.ShapeDtypeStruct((_N, _D), jnp.float32),
            jax.ShapeDtypeStruct((_N, _D), jnp.float32),
        ],
    )(x, w, c, d_in_col)


def _degnorm_body(dp_ref, d_ref):
    for h in range(2):
        deg = dp_ref[h, 0]
        for w in range(1, _NW):
            deg = deg + dp_ref[h, w]
        d_ref[h] = jnp.where(deg > 0.0, lax.rsqrt(deg), 0.0)


def _degnorm(dp):
    return pl.pallas_call(
        _degnorm_body,
        out_shape=jax.ShapeDtypeStruct((2, _CH, _CHUNK), jnp.float32),
    )(dp)


def _layer2_body(acc_ref, do_ref, di_ref, hd_ref, w_ref, c_ref,
                 t2s_ref, hd2_ref):
    h1 = jnp.maximum(do_ref[...] * (acc_ref[0] + acc_ref[1]) + hd_ref[...],
                     0.0)
    dn = (((1,), (1,)), ((), ()))
    t2s_ref[...] = di_ref[...] * lax.dot_general(
        h1, w_ref[...], dn, preferred_element_type=jnp.float32)
    hd2_ref[...] = lax.dot_general(h1, c_ref[...], dn,
                                   preferred_element_type=jnp.float32)


def _layer2(acc, d_out_col, d_in_col, hdir1, w2, c2):
    return pl.pallas_call(
        _layer2_body,
        grid=(_NBLK,),
        in_specs=[
            pl.BlockSpec((_NC, _BLK, _D), lambda i: (0, i, 0)),
            pl.BlockSpec((_BLK, 1), lambda i: (i, 0)),
            pl.BlockSpec((_BLK, 1), lambda i: (i, 0)),
            pl.BlockSpec((_BLK, _D), lambda i: (i, 0)),
            pl.BlockSpec((_D, _D), lambda i: (0, 0)),
            pl.BlockSpec((_D, _D), lambda i: (0, 0)),
        ],
        out_specs=[
            pl.BlockSpec((_BLK, _D), lambda i: (i, 0)),
            pl.BlockSpec((_BLK, _D), lambda i: (i, 0)),
        ],
        out_shape=[
            jax.ShapeDtypeStruct((_N, _D), jnp.float32),
            jax.ShapeDtypeStruct((_N, _D), jnp.float32),
        ],
    )(acc, d_out_col, d_in_col, hdir1, w2, c2)


def _final_body(acc_ref, do_ref, hd_ref, o_ref):
    o_ref[...] = do_ref[...] * (acc_ref[0] + acc_ref[1]) + hd_ref[...]


def _final(acc, d_out_col, hdir2):
    return pl.pallas_call(
        _final_body,
        grid=(_NBLK,),
        in_specs=[
            pl.BlockSpec((_NC, _BLK, _D), lambda i: (0, i, 0)),
            pl.BlockSpec((_BLK, 1), lambda i: (i, 0)),
            pl.BlockSpec((_BLK, _D), lambda i: (i, 0)),
        ],
        out_specs=pl.BlockSpec((_BLK, _D), lambda i: (i, 0)),
        out_shape=jax.ShapeDtypeStruct((_N, _D), jnp.float32),
    )(acc, d_out_col, hdir2)


def kernel(x, edge_index, W1, Wdir1, alpha1, beta1, W2, Wdir2, alpha2, beta2):
    row = edge_index[0]
    col = edge_index[1]
    row_p = row.reshape(_NW, _CH, _CHUNK)
    col_p = col.reshape(_NW, _CH, _CHUNK)

    c1 = alpha1 * W1 + beta1 * Wdir1
    c2 = alpha2 * W2 + beta2 * Wdir2

    dp = _deg_kernel(row, col)
    d = _degnorm(dp.reshape(2, _NW, _CH, _CHUNK))
    d_out_col = d[0].reshape(_N, 1)
    d_in_col = d[1].reshape(_N, 1)

    t1s, hdir1 = _mm2(x, W1, c1, d_in_col)
    acc1 = _agg_kernel(t1s, row_p, col_p)

    t2s, hdir2 = _layer2(acc1, d_out_col, d_in_col, hdir1, W2, c2)
    acc2 = _agg_kernel(t2s, row_p, col_p)

    return _final(acc2, d_out_col, hdir2)


# speedup vs baseline: 1.2792x; 1.0511x over previous
import dataclasses
import functools

import jax
import jax.numpy as jnp
from jax import lax
from jax.experimental import pallas as pl
from jax.experimental.pallas import tpu as pltpu
from jax.experimental.pallas import tpu_sc as plsc

_N = 10000
_E = 320000
_D = 128
_NC = 2
_NS = 16
_NW = _NC * _NS
_CHUNK = 125
_CH = 80
_EPW = _CH * _CHUNK
_CHS = 40
_STAGES = _CH // _CHS
_SLAB_STRIDE = 624
_SLAB = 640
_BLK = 1000
_NBLK = _N // _BLK

_mesh = plsc.VectorSubcoreMesh(core_axis_name="c", subcore_axis_name="s")

_sc_params = pltpu.CompilerParams()
if "needs_layout_passes" in pltpu.CompilerParams.__dataclass_fields__:
    _sc_params = dataclasses.replace(_sc_params, needs_layout_passes=False)


@functools.partial(
    pl.kernel,
    out_type=jax.ShapeDtypeStruct((2, _NW, _N), jnp.float32),
    mesh=_mesh,
    scratch_types=[
        pltpu.VMEM((_EPW,), jnp.int32),
        pltpu.VMEM((_EPW,), jnp.int32),
        pltpu.VMEM((_N,), jnp.float32),
        pltpu.VMEM((_N,), jnp.float32),
    ],
    compiler_params=_sc_params,
)
def _deg_kernel(row_hbm, col_hbm, out_hbm, rowv, colv, histo, histi):
    cid = lax.axis_index("c")
    sid = lax.axis_index("s")
    wid = cid * _NS + sid

    pltpu.sync_copy(row_hbm.at[pl.ds(wid * _EPW, _EPW)], rowv)
    pltpu.sync_copy(col_hbm.at[pl.ds(wid * _EPW, _EPW)], colv)

    zeros16 = jnp.zeros((16,), jnp.float32)

    @pl.loop(0, _N, step=16)
    def _zero(i):
        histo[pl.ds(i, 16)] = zeros16
        histi[pl.ds(i, 16)] = zeros16

    ones16 = jnp.ones((16,), jnp.float32)

    @pl.loop(0, _EPW, step=16)
    def _vec(k):
        plsc.addupdate_scatter(histo, [rowv[pl.ds(k, 16)]], ones16)
        plsc.addupdate_scatter(histi, [colv[pl.ds(k, 16)]], ones16)

    pltpu.sync_copy(histo, out_hbm.at[0, wid])
    pltpu.sync_copy(histi, out_hbm.at[1, wid])


@functools.partial(
    pl.kernel,
    out_type=jax.ShapeDtypeStruct((_NC, _N, _D), jnp.float32),
    mesh=_mesh,
    scratch_types=[
        pltpu.VMEM((_CHS, _CHUNK), jnp.int32),
        pltpu.VMEM((_CHS, _CHUNK), jnp.int32),
        pltpu.VMEM((_CHUNK, _D), jnp.float32),
        pltpu.VMEM((_CHUNK, _D), jnp.float32),
        pltpu.VMEM_SHARED((_N, _D), jnp.float32),
        pltpu.SemaphoreType.DMA,
        pltpu.SemaphoreType.DMA,
    ],
    compiler_params=_sc_params,
)
def _agg_kernel(t_hbm, row_hbm, col_hbm, out_hbm, rowv, colv, bufa, bufb,
                acc, sema, semb):
    cid = lax.axis_index("c")
    sid = lax.axis_index("s")
    wid = cid * _NS + sid

    zeros16 = jnp.zeros((16,), jnp.float32)

    @pl.loop(0, _CHUNK)
    def _zrow(r):
        @pl.loop(0, _D, step=16)
        def _zcol(k):
            bufa[r, pl.ds(k, 16)] = zeros16

    @pl.loop(0, _SLAB, step=64)
    def _zacc(r0):
        pltpu.sync_copy(bufa.at[pl.ds(0, 64)],
                        acc.at[pl.ds(sid * _SLAB_STRIDE + r0, 64)])

    plsc.subcore_barrier()

    @pl.loop(0, _STAGES)
    def _stage(s):
        pltpu.sync_copy(row_hbm.at[wid, pl.ds(s * _CHS, _CHS)], rowv)
        pltpu.sync_copy(col_hbm.at[wid, pl.ds(s * _CHS, _CHS)], colv)

        pltpu.async_copy(t_hbm.at[colv.at[0]], bufa, sema)

        @pl.loop(0, _CHS - 2, step=2)
        def _body(j):
            pltpu.async_copy(t_hbm.at[colv.at[j + 1]], bufb, semb)
            pltpu.make_async_copy(t_hbm.at[colv.at[j]], bufa, sema).wait()
            pltpu.sync_copy(bufa, acc.at[rowv.at[j]], add=True)
            pltpu.async_copy(t_hbm.at[colv.at[j + 2]], bufa, sema)
            pltpu.make_async_copy(t_hbm.at[colv.at[j + 1]], bufb, semb).wait()
            pltpu.sync_copy(bufb, acc.at[rowv.at[j + 1]], add=True)

        pltpu.async_copy(t_hbm.at[colv.at[_CHS - 1]], bufb, semb)
        pltpu.make_async_copy(t_hbm.at[colv.at[_CHS - 2]], bufa, sema).wait()
        pltpu.sync_copy(bufa, acc.at[rowv.at[_CHS - 2]], add=True)
        pltpu.make_async_copy(t_hbm.at[colv.at[_CHS - 1]], bufb, semb).wait()
        pltpu.sync_copy(bufb, acc.at[rowv.at[_CHS - 1]], add=True)

    plsc.subcore_barrier()

    pltpu.sync_copy(acc.at[pl.ds(sid * _SLAB_STRIDE, _SLAB)],
                    out_hbm.at[cid, pl.ds(sid * _SLAB_STRIDE, _SLAB)])


def _mm2_body(x_ref, w_ref, c_ref, di_ref, t_ref, h_ref):
    xb = x_ref[...]
    dn = (((1,), (1,)), ((), ()))
    t_ref[...] = di_ref[...] * lax.dot_general(
        xb, w_ref[...], dn, preferred_element_type=jnp.float32)
    h_ref[...] = lax.dot_general(xb, c_ref[...], dn,
                                 preferred_element_type=jnp.float32)


def _mm2(x, w, c, d_in_col):
    return pl.pallas_call(
        _mm2_body,
        grid=(_NBLK,),
        in_specs=[
            pl.BlockSpec((_BLK, _D), lambda i: (i, 0)),
            pl.BlockSpec((_D, _D), lambda i: (0, 0)),
            pl.BlockSpec((_D, _D), lambda i: (0, 0)),
            pl.BlockSpec((_BLK, 1), lambda i: (i, 0)),
        ],
        out_specs=[
            pl.BlockSpec((_BLK, _D), lambda i: (i, 0)),
            pl.BlockSpec((_BLK, _D), lambda i: (i, 0)),
        ],
        out_shape=[
            jax.ShapeDtypeStruct((_N, _D), jnp.float32),
            jax.ShapeDtypeStruct((_N, _D), jnp.float32),
        ],
    )(x, w, c, d_in_col)


def _degnorm_body(dp_ref, d_ref):
    for h in range(2):
        deg = dp_ref[h, 0]
        for w in range(1, _NW):
            deg = deg + dp_ref[h, w]
        d_ref[h] = jnp.where(deg > 0.0, lax.rsqrt(deg), 0.0)


def _degnorm(dp):
    return pl.pallas_call(
        _degnorm_body,
        out_shape=jax.ShapeDtypeStruct((2, _CH, _CHUNK), jnp.float32),
    )(dp)


def _layer2_body(acc_ref, do_ref, di_ref, hd_ref, w_ref, c_ref,
                 t2s_ref, hd2_ref):
    h1 = jnp.maximum(do_ref[...] * (acc_ref[0] + acc_ref[1]) + hd_ref[...],
                     0.0)
    dn = (((1,), (1,)), ((), ()))
    t2s_ref[...] = di_ref[...] * lax.dot_general(
        h1, w_ref[...], dn, preferred_element_type=jnp.float32)
    hd2_ref[...] = lax.dot_general(h1, c_ref[...], dn,
                                   preferred_element_type=jnp.float32)


def _layer2(acc, d_out_col, d_in_col, hdir1, w2, c2):
    return pl.pallas_call(
        _layer2_body,
        grid=(_NBLK,),
        in_specs=[
            pl.BlockSpec((_NC, _BLK, _D), lambda i: (0, i, 0)),
            pl.BlockSpec((_BLK, 1), lambda i: (i, 0)),
            pl.BlockSpec((_BLK, 1), lambda i: (i, 0)),
            pl.BlockSpec((_BLK, _D), lambda i: (i, 0)),
            pl.BlockSpec((_D, _D), lambda i: (0, 0)),
            pl.BlockSpec((_D, _D), lambda i: (0, 0)),
        ],
        out_specs=[
            pl.BlockSpec((_BLK, _D), lambda i: (i, 0)),
            pl.BlockSpec((_BLK, _D), lambda i: (i, 0)),
        ],
        out_shape=[
            jax.ShapeDtypeStruct((_N, _D), jnp.float32),
            jax.ShapeDtypeStruct((_N, _D), jnp.float32),
        ],
    )(acc, d_out_col, d_in_col, hdir1, w2, c2)


def _final_body(acc_ref, do_ref, hd_ref, o_ref):
    o_ref[...] = do_ref[...] * (acc_ref[0] + acc_ref[1]) + hd_ref[...]


def _final(acc, d_out_col, hdir2):
    return pl.pallas_call(
        _final_body,
        grid=(_NBLK,),
        in_specs=[
            pl.BlockSpec((_NC, _BLK, _D), lambda i: (0, i, 0)),
            pl.BlockSpec((_BLK, 1), lambda i: (i, 0)),
            pl.BlockSpec((_BLK, _D), lambda i: (i, 0)),
        ],
        out_specs=pl.BlockSpec((_BLK, _D), lambda i: (i, 0)),
        out_shape=jax.ShapeDtypeStruct((_N, _D), jnp.float32),
    )(acc, d_out_col, hdir2)


def kernel(x, edge_index, W1, Wdir1, alpha1, beta1, W2, Wdir2, alpha2, beta2):
    row = edge_index[0]
    col = edge_index[1]
    row_p = row.reshape(_NW, _CH, _CHUNK)
    col_p = col.reshape(_NW, _CH, _CHUNK)

    c1 = alpha1 * W1 + beta1 * Wdir1
    c2 = alpha2 * W2 + beta2 * Wdir2

    dp = _deg_kernel(row, col)
    d = _degnorm(dp.reshape(2, _NW, _CH, _CHUNK))
    d_out_col = d[0].reshape(_N, 1)
    d_in_col = d[1].reshape(_N, 1)

    t1s, hdir1 = _mm2(x, W1, c1, d_in_col)
    acc1 = _agg_kernel(t1s, row_p, col_p)

    t2s, hdir2 = _layer2(acc1, d_out_col, d_in_col, hdir1, W2, c2)
    acc2 = _agg_kernel(t2s, row_p, col_p)

    return _final(acc2, d_out_col, hdir2)


# speedup vs baseline: 1.2866x; 1.0058x over previous
import dataclasses
import functools

import jax
import jax.numpy as jnp
from jax import lax
from jax.experimental import pallas as pl
from jax.experimental.pallas import tpu as pltpu
from jax.experimental.pallas import tpu_sc as plsc

_N = 10000
_E = 320000
_D = 128
_NC = 2
_NS = 16
_NW = _NC * _NS
_CHUNK = 125
_CH = 80
_EPW = _CH * _CHUNK
_CHS = 40
_STAGES = _CH // _CHS
_SLAB_STRIDE = 624
_SLAB = 640
_BLK = 1000
_NBLK = _N // _BLK

_mesh = plsc.VectorSubcoreMesh(core_axis_name="c", subcore_axis_name="s")

_sc_params = pltpu.CompilerParams()
if "needs_layout_passes" in pltpu.CompilerParams.__dataclass_fields__:
    _sc_params = dataclasses.replace(_sc_params, needs_layout_passes=False)


@functools.partial(
    pl.kernel,
    out_type=jax.ShapeDtypeStruct((2, _NW, _N), jnp.float32),
    mesh=_mesh,
    scratch_types=[
        pltpu.VMEM((_EPW,), jnp.int32),
        pltpu.VMEM((_EPW,), jnp.int32),
        pltpu.VMEM((_N,), jnp.float32),
        pltpu.VMEM((_N,), jnp.float32),
    ],
    compiler_params=_sc_params,
)
def _deg_kernel(row_hbm, col_hbm, out_hbm, rowv, colv, histo, histi):
    cid = lax.axis_index("c")
    sid = lax.axis_index("s")
    wid = cid * _NS + sid

    pltpu.sync_copy(row_hbm.at[pl.ds(wid * _EPW, _EPW)], rowv)
    pltpu.sync_copy(col_hbm.at[pl.ds(wid * _EPW, _EPW)], colv)

    zeros16 = jnp.zeros((16,), jnp.float32)

    @pl.loop(0, _N, step=16)
    def _zero(i):
        histo[pl.ds(i, 16)] = zeros16
        histi[pl.ds(i, 16)] = zeros16

    ones16 = jnp.ones((16,), jnp.float32)

    @pl.loop(0, _EPW, step=16)
    def _vec(k):
        plsc.addupdate_scatter(histo, [rowv[pl.ds(k, 16)]], ones16)
        plsc.addupdate_scatter(histi, [colv[pl.ds(k, 16)]], ones16)

    pltpu.sync_copy(histo, out_hbm.at[0, wid])
    pltpu.sync_copy(histi, out_hbm.at[1, wid])


@functools.partial(
    pl.kernel,
    out_type=jax.ShapeDtypeStruct((_NC, _N, _D), jnp.float32),
    mesh=_mesh,
    scratch_types=[
        pltpu.VMEM((_CHS, _CHUNK), jnp.int32),
        pltpu.VMEM((_CHS, _CHUNK), jnp.int32),
        pltpu.VMEM((_CHUNK, _D), jnp.float32),
        pltpu.VMEM((_CHUNK, _D), jnp.float32),
        pltpu.VMEM_SHARED((_N, _D), jnp.float32),
        pltpu.SemaphoreType.DMA,
        pltpu.SemaphoreType.DMA,
    ],
    compiler_params=_sc_params,
)
def _agg_kernel(t_hbm, row_hbm, col_hbm, out_hbm, rowv, colv, bufa, bufb,
                acc, sema, semb):
    cid = lax.axis_index("c")
    sid = lax.axis_index("s")
    wid = cid * _NS + sid

    zeros16 = jnp.zeros((16,), jnp.float32)

    @pl.loop(0, _CHUNK)
    def _zrow(r):
        @pl.loop(0, _D, step=16)
        def _zcol(k):
            bufa[r, pl.ds(k, 16)] = zeros16

    @pl.loop(0, _SLAB, step=64)
    def _zacc(r0):
        pltpu.sync_copy(bufa.at[pl.ds(0, 64)],
                        acc.at[pl.ds(sid * _SLAB_STRIDE + r0, 64)])

    plsc.subcore_barrier()

    @pl.loop(0, _STAGES)
    def _stage(s):
        pltpu.sync_copy(row_hbm.at[wid, pl.ds(s * _CHS, _CHS)], rowv)
        pltpu.sync_copy(col_hbm.at[wid, pl.ds(s * _CHS, _CHS)], colv)

        pltpu.async_copy(t_hbm.at[colv.at[0]], bufa, sema)

        @pl.loop(0, _CHS - 2, step=2)
        def _body(j):
            pltpu.async_copy(t_hbm.at[colv.at[j + 1]], bufb, semb)
            pltpu.make_async_copy(t_hbm.at[colv.at[j]], bufa, sema).wait()
            pltpu.sync_copy(bufa, acc.at[rowv.at[j]], add=True)
            pltpu.async_copy(t_hbm.at[colv.at[j + 2]], bufa, sema)
            pltpu.make_async_copy(t_hbm.at[colv.at[j + 1]], bufb, semb).wait()
            pltpu.sync_copy(bufb, acc.at[rowv.at[j + 1]], add=True)

        pltpu.async_copy(t_hbm.at[colv.at[_CHS - 1]], bufb, semb)
        pltpu.make_async_copy(t_hbm.at[colv.at[_CHS - 2]], bufa, sema).wait()
        pltpu.sync_copy(bufa, acc.at[rowv.at[_CHS - 2]], add=True)
        pltpu.make_async_copy(t_hbm.at[colv.at[_CHS - 1]], bufb, semb).wait()
        pltpu.sync_copy(bufb, acc.at[rowv.at[_CHS - 1]], add=True)

    plsc.subcore_barrier()

    pltpu.sync_copy(acc.at[pl.ds(sid * _SLAB_STRIDE, _SLAB)],
                    out_hbm.at[cid, pl.ds(sid * _SLAB_STRIDE, _SLAB)])


def _mm2_body(x_ref, w_ref, c_ref, di_ref, t_ref, h_ref):
    xb = x_ref[...].astype(jnp.bfloat16)
    dn = (((1,), (1,)), ((), ()))
    t_ref[...] = di_ref[...] * lax.dot_general(
        xb, w_ref[...].astype(jnp.bfloat16), dn,
        preferred_element_type=jnp.float32)
    h_ref[...] = lax.dot_general(xb, c_ref[...].astype(jnp.bfloat16), dn,
                                 preferred_element_type=jnp.float32)


def _mm2(x, w, c, d_in_col):
    return pl.pallas_call(
        _mm2_body,
        grid=(_NBLK,),
        in_specs=[
            pl.BlockSpec((_BLK, _D), lambda i: (i, 0)),
            pl.BlockSpec((_D, _D), lambda i: (0, 0)),
            pl.BlockSpec((_D, _D), lambda i: (0, 0)),
            pl.BlockSpec((_BLK, 1), lambda i: (i, 0)),
        ],
        out_specs=[
            pl.BlockSpec((_BLK, _D), lambda i: (i, 0)),
            pl.BlockSpec((_BLK, _D), lambda i: (i, 0)),
        ],
        out_shape=[
            jax.ShapeDtypeStruct((_N, _D), jnp.float32),
            jax.ShapeDtypeStruct((_N, _D), jnp.float32),
        ],
    )(x, w, c, d_in_col)


def _degnorm_body(dp_ref, d_ref):
    for h in range(2):
        deg = dp_ref[h, 0]
        for w in range(1, _NW):
            deg = deg + dp_ref[h, w]
        d_ref[h] = jnp.where(deg > 0.0, lax.rsqrt(deg), 0.0)


def _degnorm(dp):
    return pl.pallas_call(
        _degnorm_body,
        out_shape=jax.ShapeDtypeStruct((2, _CH, _CHUNK), jnp.float32),
    )(dp)


def _layer2_body(acc_ref, do_ref, di_ref, hd_ref, w_ref, c_ref,
                 t2s_ref, hd2_ref):
    h1 = jnp.maximum(do_ref[...] * (acc_ref[0] + acc_ref[1]) + hd_ref[...],
                     0.0).astype(jnp.bfloat16)
    dn = (((1,), (1,)), ((), ()))
    t2s_ref[...] = di_ref[...] * lax.dot_general(
        h1, w_ref[...].astype(jnp.bfloat16), dn,
        preferred_element_type=jnp.float32)
    hd2_ref[...] = lax.dot_general(h1, c_ref[...].astype(jnp.bfloat16), dn,
                                   preferred_element_type=jnp.float32)


def _layer2(acc, d_out_col, d_in_col, hdir1, w2, c2):
    return pl.pallas_call(
        _layer2_body,
        grid=(_NBLK,),
        in_specs=[
            pl.BlockSpec((_NC, _BLK, _D), lambda i: (0, i, 0)),
            pl.BlockSpec((_BLK, 1), lambda i: (i, 0)),
            pl.BlockSpec((_BLK, 1), lambda i: (i, 0)),
            pl.BlockSpec((_BLK, _D), lambda i: (i, 0)),
            pl.BlockSpec((_D, _D), lambda i: (0, 0)),
            pl.BlockSpec((_D, _D), lambda i: (0, 0)),
        ],
        out_specs=[
            pl.BlockSpec((_BLK, _D), lambda i: (i, 0)),
            pl.BlockSpec((_BLK, _D), lambda i: (i, 0)),
        ],
        out_shape=[
            jax.ShapeDtypeStruct((_N, _D), jnp.float32),
            jax.ShapeDtypeStruct((_N, _D), jnp.float32),
        ],
    )(acc, d_out_col, d_in_col, hdir1, w2, c2)


def _final_body(acc_ref, do_ref, hd_ref, o_ref):
    o_ref[...] = do_ref[...] * (acc_ref[0] + acc_ref[1]) + hd_ref[...]


def _final(acc, d_out_col, hdir2):
    return pl.pallas_call(
        _final_body,
        grid=(_NBLK,),
        in_specs=[
            pl.BlockSpec((_NC, _BLK, _D), lambda i: (0, i, 0)),
            pl.BlockSpec((_BLK, 1), lambda i: (i, 0)),
            pl.BlockSpec((_BLK, _D), lambda i: (i, 0)),
        ],
        out_specs=pl.BlockSpec((_BLK, _D), lambda i: (i, 0)),
        out_shape=jax.ShapeDtypeStruct((_N, _D), jnp.float32),
    )(acc, d_out_col, hdir2)


def kernel(x, edge_index, W1, Wdir1, alpha1, beta1, W2, Wdir2, alpha2, beta2):
    row = edge_index[0]
    col = edge_index[1]
    row_p = row.reshape(_NW, _CH, _CHUNK)
    col_p = col.reshape(_NW, _CH, _CHUNK)

    c1 = alpha1 * W1 + beta1 * Wdir1
    c2 = alpha2 * W2 + beta2 * Wdir2

    dp = _deg_kernel(row, col)
    d = _degnorm(dp.reshape(2, _NW, _CH, _CHUNK))
    d_out_col = d[0].reshape(_N, 1)
    d_in_col = d[1].reshape(_N, 1)

    t1s, hdir1 = _mm2(x, W1, c1, d_in_col)
    acc1 = _agg_kernel(t1s, row_p, col_p)

    t2s, hdir2 = _layer2(acc1, d_out_col, d_in_col, hdir1, W2, c2)
    acc2 = _agg_kernel(t2s, row_p, col_p)

    return _final(acc2, d_out_col, hdir2)


# speedup vs baseline: 1.3044x; 1.0138x over previous
import dataclasses
import functools

import jax
import jax.numpy as jnp
from jax import lax
from jax.experimental import pallas as pl
from jax.experimental.pallas import tpu as pltpu
from jax.experimental.pallas import tpu_sc as plsc

_N = 10000
_E = 320000
_D = 128
_NC = 2
_NS = 16
_NW = _NC * _NS
_CHUNK = 125
_CH = 80
_EPW = _CH * _CHUNK
_CHS = 40
_STAGES = _CH // _CHS
_SLAB_STRIDE = 624
_SLAB = 640
_BLK = 2000
_NBLK = _N // _BLK

_mesh = plsc.VectorSubcoreMesh(core_axis_name="c", subcore_axis_name="s")

_sc_params = pltpu.CompilerParams()
if "needs_layout_passes" in pltpu.CompilerParams.__dataclass_fields__:
    _sc_params = dataclasses.replace(_sc_params, needs_layout_passes=False)


@functools.partial(
    pl.kernel,
    out_type=jax.ShapeDtypeStruct((2, _NW, _N), jnp.float32),
    mesh=_mesh,
    scratch_types=[
        pltpu.VMEM((_EPW,), jnp.int32),
        pltpu.VMEM((_EPW,), jnp.int32),
        pltpu.VMEM((_N,), jnp.float32),
        pltpu.VMEM((_N,), jnp.float32),
    ],
    compiler_params=_sc_params,
)
def _deg_kernel(row_hbm, col_hbm, out_hbm, rowv, colv, histo, histi):
    cid = lax.axis_index("c")
    sid = lax.axis_index("s")
    wid = cid * _NS + sid

    pltpu.sync_copy(row_hbm.at[pl.ds(wid * _EPW, _EPW)], rowv)
    pltpu.sync_copy(col_hbm.at[pl.ds(wid * _EPW, _EPW)], colv)

    zeros16 = jnp.zeros((16,), jnp.float32)

    @pl.loop(0, _N, step=16)
    def _zero(i):
        histo[pl.ds(i, 16)] = zeros16
        histi[pl.ds(i, 16)] = zeros16

    ones16 = jnp.ones((16,), jnp.float32)

    @pl.loop(0, _EPW, step=16)
    def _vec(k):
        plsc.addupdate_scatter(histo, [rowv[pl.ds(k, 16)]], ones16)
        plsc.addupdate_scatter(histi, [colv[pl.ds(k, 16)]], ones16)

    pltpu.sync_copy(histo, out_hbm.at[0, wid])
    pltpu.sync_copy(histi, out_hbm.at[1, wid])


@functools.partial(
    pl.kernel,
    out_type=jax.ShapeDtypeStruct((_NC, _N, _D), jnp.float32),
    mesh=_mesh,
    scratch_types=[
        pltpu.VMEM((_CHS, _CHUNK), jnp.int32),
        pltpu.VMEM((_CHS, _CHUNK), jnp.int32),
        pltpu.VMEM((_CHUNK, _D), jnp.float32),
        pltpu.VMEM((_CHUNK, _D), jnp.float32),
        pltpu.VMEM_SHARED((_N, _D), jnp.float32),
        pltpu.SemaphoreType.DMA,
        pltpu.SemaphoreType.DMA,
    ],
    compiler_params=_sc_params,
)
def _agg_kernel(t_hbm, row_hbm, col_hbm, out_hbm, rowv, colv, bufa, bufb,
                acc, sema, semb):
    cid = lax.axis_index("c")
    sid = lax.axis_index("s")
    wid = cid * _NS + sid

    zeros16 = jnp.zeros((16,), jnp.float32)

    @pl.loop(0, _CHUNK)
    def _zrow(r):
        @pl.loop(0, _D, step=16)
        def _zcol(k):
            bufa[r, pl.ds(k, 16)] = zeros16

    @pl.loop(0, _SLAB, step=64)
    def _zacc(r0):
        pltpu.sync_copy(bufa.at[pl.ds(0, 64)],
                        acc.at[pl.ds(sid * _SLAB_STRIDE + r0, 64)])

    plsc.subcore_barrier()

    @pl.loop(0, _STAGES)
    def _stage(s):
        pltpu.sync_copy(row_hbm.at[wid, pl.ds(s * _CHS, _CHS)], rowv)
        pltpu.sync_copy(col_hbm.at[wid, pl.ds(s * _CHS, _CHS)], colv)

        pltpu.async_copy(t_hbm.at[colv.at[0]], bufa, sema)

        @pl.loop(0, _CHS - 2, step=2)
        def _body(j):
            pltpu.async_copy(t_hbm.at[colv.at[j + 1]], bufb, semb)
            pltpu.make_async_copy(t_hbm.at[colv.at[j]], bufa, sema).wait()
            pltpu.sync_copy(bufa, acc.at[rowv.at[j]], add=True)
            pltpu.async_copy(t_hbm.at[colv.at[j + 2]], bufa, sema)
            pltpu.make_async_copy(t_hbm.at[colv.at[j + 1]], bufb, semb).wait()
            pltpu.sync_copy(bufb, acc.at[rowv.at[j + 1]], add=True)

        pltpu.async_copy(t_hbm.at[colv.at[_CHS - 1]], bufb, semb)
        pltpu.make_async_copy(t_hbm.at[colv.at[_CHS - 2]], bufa, sema).wait()
        pltpu.sync_copy(bufa, acc.at[rowv.at[_CHS - 2]], add=True)
        pltpu.make_async_copy(t_hbm.at[colv.at[_CHS - 1]], bufb, semb).wait()
        pltpu.sync_copy(bufb, acc.at[rowv.at[_CHS - 1]], add=True)

    plsc.subcore_barrier()

    pltpu.sync_copy(acc.at[pl.ds(sid * _SLAB_STRIDE, _SLAB)],
                    out_hbm.at[cid, pl.ds(sid * _SLAB_STRIDE, _SLAB)])


def _mm2_body(x_ref, w_ref, c_ref, di_ref, t_ref, h_ref):
    xb = x_ref[...].astype(jnp.bfloat16)
    dn = (((1,), (1,)), ((), ()))
    t_ref[...] = di_ref[...] * lax.dot_general(
        xb, w_ref[...].astype(jnp.bfloat16), dn,
        preferred_element_type=jnp.float32)
    h_ref[...] = lax.dot_general(xb, c_ref[...].astype(jnp.bfloat16), dn,
                                 preferred_element_type=jnp.float32)


def _mm2(x, w, c, d_in_col):
    return pl.pallas_call(
        _mm2_body,
        grid=(_NBLK,),
        in_specs=[
            pl.BlockSpec((_BLK, _D), lambda i: (i, 0)),
            pl.BlockSpec((_D, _D), lambda i: (0, 0)),
            pl.BlockSpec((_D, _D), lambda i: (0, 0)),
            pl.BlockSpec((_BLK, 1), lambda i: (i, 0)),
        ],
        out_specs=[
            pl.BlockSpec((_BLK, _D), lambda i: (i, 0)),
            pl.BlockSpec((_BLK, _D), lambda i: (i, 0)),
        ],
        out_shape=[
            jax.ShapeDtypeStruct((_N, _D), jnp.float32),
            jax.ShapeDtypeStruct((_N, _D), jnp.float32),
        ],
    )(x, w, c, d_in_col)


def _degnorm_body(dp_ref, d_ref):
    for h in range(2):
        deg = dp_ref[h, 0]
        for w in range(1, _NW):
            deg = deg + dp_ref[h, w]
        d_ref[h] = jnp.where(deg > 0.0, lax.rsqrt(deg), 0.0)


def _degnorm(dp):
    return pl.pallas_call(
        _degnorm_body,
        out_shape=jax.ShapeDtypeStruct((2, _CH, _CHUNK), jnp.float32),
    )(dp)


def _layer2_body(acc_ref, do_ref, di_ref, hd_ref, w_ref, c_ref,
                 t2s_ref, hd2_ref):
    h1 = jnp.maximum(do_ref[...] * (acc_ref[0] + acc_ref[1]) + hd_ref[...],
                     0.0).astype(jnp.bfloat16)
    dn = (((1,), (1,)), ((), ()))
    t2s_ref[...] = di_ref[...] * lax.dot_general(
        h1, w_ref[...].astype(jnp.bfloat16), dn,
        preferred_element_type=jnp.float32)
    hd2_ref[...] = lax.dot_general(h1, c_ref[...].astype(jnp.bfloat16), dn,
                                   preferred_element_type=jnp.float32)


def _layer2(acc, d_out_col, d_in_col, hdir1, w2, c2):
    return pl.pallas_call(
        _layer2_body,
        grid=(_NBLK,),
        in_specs=[
            pl.BlockSpec((_NC, _BLK, _D), lambda i: (0, i, 0)),
            pl.BlockSpec((_BLK, 1), lambda i: (i, 0)),
            pl.BlockSpec((_BLK, 1), lambda i: (i, 0)),
            pl.BlockSpec((_BLK, _D), lambda i: (i, 0)),
            pl.BlockSpec((_D, _D), lambda i: (0, 0)),
            pl.BlockSpec((_D, _D), lambda i: (0, 0)),
        ],
        out_specs=[
            pl.BlockSpec((_BLK, _D), lambda i: (i, 0)),
            pl.BlockSpec((_BLK, _D), lambda i: (i, 0)),
        ],
        out_shape=[
            jax.ShapeDtypeStruct((_N, _D), jnp.float32),
            jax.ShapeDtypeStruct((_N, _D), jnp.float32),
        ],
    )(acc, d_out_col, d_in_col, hdir1, w2, c2)


def _final_body(acc_ref, do_ref, hd_ref, o_ref):
    o_ref[...] = do_ref[...] * (acc_ref[0] + acc_ref[1]) + hd_ref[...]


def _final(acc, d_out_col, hdir2):
    return pl.pallas_call(
        _final_body,
        grid=(_NBLK,),
        in_specs=[
            pl.BlockSpec((_NC, _BLK, _D), lambda i: (0, i, 0)),
            pl.BlockSpec((_BLK, 1), lambda i: (i, 0)),
            pl.BlockSpec((_BLK, _D), lambda i: (i, 0)),
        ],
        out_specs=pl.BlockSpec((_BLK, _D), lambda i: (i, 0)),
        out_shape=jax.ShapeDtypeStruct((_N, _D), jnp.float32),
    )(acc, d_out_col, hdir2)


def kernel(x, edge_index, W1, Wdir1, alpha1, beta1, W2, Wdir2, alpha2, beta2):
    row = edge_index[0]
    col = edge_index[1]
    row_p = row.reshape(_NW, _CH, _CHUNK)
    col_p = col.reshape(_NW, _CH, _CHUNK)

    c1 = alpha1 * W1 + beta1 * Wdir1
    c2 = alpha2 * W2 + beta2 * Wdir2

    dp = _deg_kernel(row, col)
    d = _degnorm(dp.reshape(2, _NW, _CH, _CHUNK))
    d_out_col = d[0].reshape(_N, 1)
    d_in_col = d[1].reshape(_N, 1)

    t1s, hdir1 = _mm2(x, W1, c1, d_in_col)
    acc1 = _agg_kernel(t1s, row_p, col_p)

    t2s, hdir2 = _layer2(acc1, d_out_col, d_in_col, hdir1, W2, c2)
    acc2 = _agg_kernel(t2s, row_p, col_p)

    return _final(acc2, d_out_col, hdir2)
